# Initial kernel scaffold; baseline (speedup 1.0000x reference)
import jax
import jax.numpy as jnp
from jax.experimental import pallas as pl


def kernel(x):
    raise NotImplementedError("write your pallas kernel here")


# speedup vs baseline: 1.8903x; 1.8903x over previous
import jax
import jax.numpy as jnp
from jax import lax
from jax.experimental import pallas as pl

_ROWS = 16384
_G = 256
_BLOCK_ROWS = 1024


def _body(x_ref, o_ref):
    xb = x_ref[...]
    m = jnp.max(xb, axis=1, keepdims=True)
    eq = xb == m
    iota = lax.broadcasted_iota(jnp.int32, xb.shape, 1)
    imin = jnp.min(jnp.where(eq, iota, _G), axis=1, keepdims=True)
    o_ref[...] = (iota == imin).astype(jnp.float32)


def kernel(x):
    xr = x.reshape(_ROWS, _G)
    out = pl.pallas_call(
        _body,
        grid=(_ROWS // _BLOCK_ROWS,),
        in_specs=[pl.BlockSpec((_BLOCK_ROWS, _G), lambda i: (i, 0))],
        out_specs=pl.BlockSpec((_BLOCK_ROWS, _G), lambda i: (i, 0)),
        out_shape=jax.ShapeDtypeStruct((_ROWS, _G), jnp.float32),
    )(xr)
    return out.reshape(x.shape)


# speedup vs baseline: 4.8666x; 2.5746x over previous
import jax
import jax.numpy as jnp
from jax import lax
from jax.experimental import pallas as pl

_R = 128
_C = 32768
_G = 256
_BLOCK_C = 4096


def _body(x_ref, o_ref):
    for k in range(_BLOCK_C // _G):
        xb = x_ref[:, k * _G:(k + 1) * _G]
        m = jnp.max(xb, axis=1, keepdims=True)
        eq = xb == m
        iota = lax.broadcasted_iota(jnp.int32, (_R, _G), 1)
        imin = jnp.min(jnp.where(eq, iota, _G), axis=1, keepdims=True)
        o_ref[:, k * _G:(k + 1) * _G] = (iota == imin).astype(jnp.float32)


def kernel(x):
    return pl.pallas_call(
        _body,
        grid=(_C // _BLOCK_C,),
        in_specs=[pl.BlockSpec((_R, _BLOCK_C), lambda j: (0, j))],
        out_specs=pl.BlockSpec((_R, _BLOCK_C), lambda j: (0, j)),
        out_shape=jax.ShapeDtypeStruct((_R, _C), jnp.float32),
    )(x)
